# baseline (speedup 1.0000x reference)
import functools

import jax
import jax.numpy as jnp
import numpy as np
from jax import lax
from jax.experimental import pallas as pl
from jax.experimental.pallas import tpu as pltpu
import jax.experimental.pallas.tpu_sc as plsc

M = 512
N = M * M
KDIAG = 2
B, C = 2, 128
BC = B * C

_ti_r, _ti_c = np.triu_indices(M, k=KDIAG)
_GIDX = (_ti_r * M + _ti_c).astype(np.int64)
OUTLEN = int(_GIDX.shape[0])

OUTCAP = 4096
ROWCAP8 = 64

_seg_len = np.maximum(M - KDIAG - np.arange(M), 0)
_off = np.concatenate([[0], np.cumsum(_seg_len)])


def _row_of(p):
    return int(np.searchsorted(_off, p, side="right") - 1)


def _win(r0, r1):
    a = r0 & ~7
    n8 = -(-(r1 + 1 - a) // 8) * 8
    return a, n8


def _build_plan():
    chunks = []
    p = 0
    while p < OUTLEN:
        s = min(OUTCAP, OUTLEN - p)
        while True:
            r0, r1 = _row_of(p), _row_of(p + s - 1)
            fa, fn = _win(r0, r1)
            ra, rn = _win(511 - r1, 511 - r0)
            n8c = max(fn, rn)
            if n8c <= ROWCAP8:
                break
            s = (s - 1) // 128 * 128
            assert s > 0
        final = (p + s == OUTLEN)
        spad = -(-s // 128) * 128
        if not final:
            assert s % 128 == 0
        fa = min(fa, M - n8c)
        ra = min(ra, M - n8c)
        wf = M - (r0 + 2) // 128 * 128
        wr = -(-(510 - r0) // 128) * 128
        w = min(M, max(wf, wr))
        chunks.append(dict(p=p, s=s, spad=spad, n8=n8c, w=w,
                           fwd_start=fa, flip_start=ra, final=final))
        p += s
    tot = sum(ch["spad"] for ch in chunks)
    tabs = np.zeros((2, 2, 1, tot), np.int32)
    ioff = 0
    for ch in chunks:
        g = _GIDX[ch["p"]: ch["p"] + ch["s"]]
        r, c = g // M, g % M
        sl = slice(ioff, ioff + ch["s"])
        tabs[1, 0, 0, sl] = r - ch["fwd_start"]
        tabs[1, 1, 0, sl] = c - (M - ch["w"])
        tabs[0, 0, 0, sl] = (511 - r) - ch["flip_start"]
        tabs[0, 1, 0, sl] = 511 - c
        assert tabs[1, 1, 0, sl].min() >= 0 and tabs[0, 1, 0, sl].max() < ch["w"]
        ch["idx_off"] = ioff
        ioff += ch["spad"]
    assert tabs[:, 0].max() < ROWCAP8 and tabs.min() >= 0
    return chunks, tabs


_CHUNKS, _TABS = _build_plan()
_NCH = len(_CHUNKS)

NC, NS = 2, 16
NW = NC * NS
PAIRS_PER_W = BC // NW


def _sc_body(in_ref, idx_ref, flags_ref, out_ref,
             win_v, ri_v, ci_v, out8_v, flag_v, sems, isems, osem):
    wid = lax.axis_index("s") * NC + lax.axis_index("c")
    b = wid // NS
    ch0 = (wid % NS) * PAIRS_PER_W

    pltpu.sync_copy(flags_ref, flag_v)
    fv = flag_v[...]
    lane = lax.broadcasted_iota(jnp.int32, (16,), 0)
    flag = jnp.sum(jnp.where(lane == b, fv, 0))

    prev_out = [None]

    for ch in _CHUNKS:
        spad, s, n8 = ch["spad"], ch["s"], ch["n8"]
        idx_cp = [
            pltpu.make_async_copy(
                idx_ref.at[flag, t, 0, pl.ds(ch["idx_off"], spad)],
                (ri_v if t == 0 else ci_v).at[pl.ds(0, spad)],
                isems.at[t])
            for t in range(2)
        ]
        for cp in idx_cp:
            cp.start()
        rs = pl.multiple_of(
            flag * ch["fwd_start"] + (1 - flag) * ch["flip_start"], 8)
        w = ch["w"]
        c0 = pl.multiple_of(flag * (M - w), 128)

        def start_win(k, par, n8=n8, rs=rs, w=w, c0=c0):
            pltpu.make_async_copy(
                in_ref.at[b, ch0 + k, pl.ds(rs, n8), pl.ds(c0, w)],
                win_v.at[par, pl.ds(0, n8), pl.ds(0, w)],
                sems.at[par]).start()

        start_win(0, 0)
        for cp in idx_cp:
            cp.wait()
        if prev_out[0] is not None:
            prev_out[0].wait()

        def chan_body(k, carry, n8=n8, spad=spad, rs=rs, w=w, c0=c0):
            par = lax.rem(k, 2)
            pltpu.make_async_copy(
                in_ref.at[b, ch0 + k, pl.ds(rs, n8), pl.ds(c0, w)],
                win_v.at[par, pl.ds(0, n8), pl.ds(0, w)],
                sems.at[par]).wait()

            @pl.when(k < PAIRS_PER_W - 1)
            def _():
                start_win(k + 1, 1 - par)

            win = win_v.at[par]

            @plsc.parallel_loop(0, spad, step=16, unroll=16)
            def gbody(j, k=k):
                riv = ri_v[pl.ds(j, 16)]
                civ = ci_v[pl.ds(j, 16)]
                out8_v[k, pl.ds(j, 16)] = plsc.load_gather(win, [riv, civ])

            return carry

        lax.fori_loop(0, PAIRS_PER_W, chan_body, 0)
        if ch["final"]:
            pstart = pl.multiple_of(ch["p"] + 0 * flag, 128)
        else:
            pstart = ch["p"]
        ocp = pltpu.make_async_copy(
            out8_v.at[:, pl.ds(0, spad)],
            out_ref.at[b, pl.ds(ch0, 8), pl.ds(pstart, spad)],
            osem)
        ocp.start()
        prev_out[0] = ocp
    prev_out[0].wait()


JB = 512
_NJB = -(-OUTLEN // JB)


def _tc_transpose_body(y_ref, z_ref):
    x = y_ref[...].reshape(B * C, JB)
    z_ref[...] = x.T.reshape(JB, B, C)


def _to_channel_minor(y):
    z = pl.pallas_call(
        _tc_transpose_body,
        grid=(_NJB,),
        in_specs=[pl.BlockSpec((B, C, JB), lambda j: (0, 0, j))],
        out_specs=pl.BlockSpec((JB, B, C), lambda j: (j, 0, 0)),
        out_shape=jax.ShapeDtypeStruct((OUTLEN, B, C), jnp.float32),
    )(y)
    return jnp.transpose(z, (1, 2, 0))


def kernel(inputs, reverse_complement_flags):
    flags16 = jnp.zeros((16,), jnp.int32).at[:B].set(
        reverse_complement_flags.astype(jnp.int32))
    idx_tab = jnp.asarray(_TABS)

    mesh = plsc.VectorSubcoreMesh(core_axis_name="c", subcore_axis_name="s",
                                  num_cores=NC, num_subcores=NS)
    fn = pl.kernel(
        _sc_body,
        out_type=jax.ShapeDtypeStruct((B, C, OUTLEN), jnp.float32),
        mesh=mesh,
        scratch_types=[
            pltpu.VMEM((2, ROWCAP8, M), jnp.float32),
            pltpu.VMEM((OUTCAP,), jnp.int32),
            pltpu.VMEM((OUTCAP,), jnp.int32),
            pltpu.VMEM((PAIRS_PER_W, OUTCAP), jnp.float32),
            pltpu.VMEM((16,), jnp.int32),
            pltpu.SemaphoreType.DMA((2,)),
            pltpu.SemaphoreType.DMA((2,)),
            pltpu.SemaphoreType.DMA,
        ],
        compiler_params=pltpu.CompilerParams(disable_bounds_checks=True,
                                             needs_layout_passes=False),
    )
    return _to_channel_minor(fn(inputs, idx_tab, flags16))


# speedup vs baseline: 1.0024x; 1.0024x over previous
import functools

import jax
import jax.numpy as jnp
import numpy as np
from jax import lax
from jax.experimental import pallas as pl
from jax.experimental.pallas import tpu as pltpu
import jax.experimental.pallas.tpu_sc as plsc

M = 512
N = M * M
KDIAG = 2
B, C = 2, 128
BC = B * C

_ti_r, _ti_c = np.triu_indices(M, k=KDIAG)
_GIDX = (_ti_r * M + _ti_c).astype(np.int64)
OUTLEN = int(_GIDX.shape[0])

OUTCAP = 4096
ROWCAP8 = 64

_seg_len = np.maximum(M - KDIAG - np.arange(M), 0)
_off = np.concatenate([[0], np.cumsum(_seg_len)])


def _row_of(p):
    return int(np.searchsorted(_off, p, side="right") - 1)


def _win(r0, r1):
    a = r0 & ~7
    n8 = -(-(r1 + 1 - a) // 8) * 8
    return a, n8


def _build_plan():
    chunks = []
    p = 0
    while p < OUTLEN:
        s = min(OUTCAP, OUTLEN - p)
        while True:
            r0, r1 = _row_of(p), _row_of(p + s - 1)
            fa, fn = _win(r0, r1)
            ra, rn = _win(511 - r1, 511 - r0)
            n8c = max(fn, rn)
            if n8c <= ROWCAP8:
                break
            s = (s - 1) // 128 * 128
            assert s > 0
        final = (p + s == OUTLEN)
        spad = -(-s // 128) * 128
        if not final:
            assert s % 128 == 0
        fa = min(fa, M - n8c)
        ra = min(ra, M - n8c)
        wf = M - (r0 + 2) // 128 * 128
        wr = -(-(510 - r0) // 128) * 128
        w = min(M, max(wf, wr))
        chunks.append(dict(p=p, s=s, spad=spad, n8=n8c, w=w,
                           fwd_start=fa, flip_start=ra, final=final))
        p += s
    tot = sum(ch["spad"] for ch in chunks)
    tabs = np.zeros((2, 2, 1, tot), np.int32)
    ioff = 0
    for ch in chunks:
        g = _GIDX[ch["p"]: ch["p"] + ch["s"]]
        r, c = g // M, g % M
        sl = slice(ioff, ioff + ch["s"])
        tabs[1, 0, 0, sl] = r - ch["fwd_start"]
        tabs[1, 1, 0, sl] = c - (M - ch["w"])
        tabs[0, 0, 0, sl] = (511 - r) - ch["flip_start"]
        tabs[0, 1, 0, sl] = 511 - c
        assert tabs[1, 1, 0, sl].min() >= 0 and tabs[0, 1, 0, sl].max() < ch["w"]
        ch["idx_off"] = ioff
        ioff += ch["spad"]
    assert tabs[:, 0].max() < ROWCAP8 and tabs.min() >= 0
    return chunks, tabs


_CHUNKS, _TABS = _build_plan()
_NCH = len(_CHUNKS)

NC, NS = 2, 16
NW = NC * NS
PAIRS_PER_W = BC // NW


def _sc_body(in_ref, idx_ref, flags_ref, out_ref,
             win_v, ri_v, ci_v, out8_v, flag_v, sems, isems, osem):
    wid = lax.axis_index("s") * NC + lax.axis_index("c")
    b = wid // NS
    ch0 = (wid % NS) * PAIRS_PER_W

    pltpu.sync_copy(flags_ref, flag_v)
    fv = flag_v[...]
    lane = lax.broadcasted_iota(jnp.int32, (16,), 0)
    flag = jnp.sum(jnp.where(lane == b, fv, 0))

    prev_out = [None]

    for ch in _CHUNKS:
        spad, s, n8 = ch["spad"], ch["s"], ch["n8"]
        idx_cp = [
            pltpu.make_async_copy(
                idx_ref.at[flag, t, 0, pl.ds(ch["idx_off"], spad)],
                (ri_v if t == 0 else ci_v).at[pl.ds(0, spad)],
                isems.at[t])
            for t in range(2)
        ]
        for cp in idx_cp:
            cp.start()
        rs = pl.multiple_of(
            flag * ch["fwd_start"] + (1 - flag) * ch["flip_start"], 8)
        w = ch["w"]
        c0 = pl.multiple_of(flag * (M - w), 128)

        def start_win(k, par, n8=n8, rs=rs, w=w, c0=c0):
            pltpu.make_async_copy(
                in_ref.at[b, ch0 + k, pl.ds(rs, n8), pl.ds(c0, w)],
                win_v.at[par, pl.ds(0, n8), pl.ds(0, w)],
                sems.at[par]).start()

        start_win(0, 0)
        for cp in idx_cp:
            cp.wait()
        if prev_out[0] is not None:
            prev_out[0].wait()

        def chan_body(k, carry, n8=n8, spad=spad, rs=rs, w=w, c0=c0):
            par = lax.rem(k, 2)
            pltpu.make_async_copy(
                in_ref.at[b, ch0 + k, pl.ds(rs, n8), pl.ds(c0, w)],
                win_v.at[par, pl.ds(0, n8), pl.ds(0, w)],
                sems.at[par]).wait()

            @pl.when(k < PAIRS_PER_W - 1)
            def _():
                start_win(k + 1, 1 - par)

            win = win_v.at[par]

            @plsc.parallel_loop(0, spad, step=16, unroll=16)
            def gbody(j, k=k):
                riv = ri_v[pl.ds(j, 16)]
                civ = ci_v[pl.ds(j, 16)]
                out8_v[k, pl.ds(j, 16)] = plsc.load_gather(win, [riv, civ])

            return carry

        lax.fori_loop(0, PAIRS_PER_W, chan_body, 0)
        if ch["final"]:
            pstart = pl.multiple_of(ch["p"] + 0 * flag, 128)
        else:
            pstart = ch["p"]
        ocp = pltpu.make_async_copy(
            out8_v.at[:, pl.ds(0, spad)],
            out_ref.at[b, pl.ds(ch0, 8), pl.ds(pstart, spad)],
            osem)
        ocp.start()
        prev_out[0] = ocp
    prev_out[0].wait()


JB = 512
_NJB = -(-OUTLEN // JB)


def _tc_transpose_body(y_ref, z_ref):
    x = y_ref[...].reshape(B * C, JB)
    z_ref[...] = x.T.reshape(JB, B, C)


def _to_channel_minor(y):
    z = pl.pallas_call(
        _tc_transpose_body,
        grid=(_NJB,),
        in_specs=[pl.BlockSpec((B, C, JB), lambda j: (0, 0, j))],
        out_specs=pl.BlockSpec((JB, B, C), lambda j: (j, 0, 0)),
        out_shape=jax.ShapeDtypeStruct((OUTLEN, B, C), jnp.float32),
        compiler_params=pltpu.CompilerParams(
            dimension_semantics=("arbitrary",)),
    )(y)
    return jnp.transpose(z, (1, 2, 0))


def kernel(inputs, reverse_complement_flags):
    flags16 = jnp.zeros((16,), jnp.int32).at[:B].set(
        reverse_complement_flags.astype(jnp.int32))
    idx_tab = jnp.asarray(_TABS)

    mesh = plsc.VectorSubcoreMesh(core_axis_name="c", subcore_axis_name="s",
                                  num_cores=NC, num_subcores=NS)
    fn = pl.kernel(
        _sc_body,
        out_type=jax.ShapeDtypeStruct((B, C, OUTLEN), jnp.float32),
        mesh=mesh,
        scratch_types=[
            pltpu.VMEM((2, ROWCAP8, M), jnp.float32),
            pltpu.VMEM((OUTCAP,), jnp.int32),
            pltpu.VMEM((OUTCAP,), jnp.int32),
            pltpu.VMEM((PAIRS_PER_W, OUTCAP), jnp.float32),
            pltpu.VMEM((16,), jnp.int32),
            pltpu.SemaphoreType.DMA((2,)),
            pltpu.SemaphoreType.DMA((2,)),
            pltpu.SemaphoreType.DMA,
        ],
        compiler_params=pltpu.CompilerParams(disable_bounds_checks=True,
                                             needs_layout_passes=False),
    )
    return _to_channel_minor(fn(inputs, idx_tab, flags16))


# speedup vs baseline: 1.1184x; 1.1158x over previous
import functools

import jax
import jax.numpy as jnp
import numpy as np
from jax import lax
from jax.experimental import pallas as pl
from jax.experimental.pallas import tpu as pltpu
import jax.experimental.pallas.tpu_sc as plsc

M = 512
N = M * M
KDIAG = 2
B, C = 2, 128
BC = B * C

_ti_r, _ti_c = np.triu_indices(M, k=KDIAG)
_GIDX = (_ti_r * M + _ti_c).astype(np.int64)
OUTLEN = int(_GIDX.shape[0])

OUTCAP = 4096
ROWCAP8 = 64

_seg_len = np.maximum(M - KDIAG - np.arange(M), 0)
_off = np.concatenate([[0], np.cumsum(_seg_len)])


def _row_of(p):
    return int(np.searchsorted(_off, p, side="right") - 1)


def _win(r0, r1):
    a = r0 & ~7
    n8 = -(-(r1 + 1 - a) // 8) * 8
    return a, n8


def _build_plan():
    chunks = []
    p = 0
    while p < OUTLEN:
        s = min(OUTCAP, OUTLEN - p)
        while True:
            r0, r1 = _row_of(p), _row_of(p + s - 1)
            fa, fn = _win(r0, r1)
            ra, rn = _win(511 - r1, 511 - r0)
            n8c = max(fn, rn)
            if n8c <= ROWCAP8:
                break
            s = (s - 1) // 128 * 128
            assert s > 0
        final = (p + s == OUTLEN)
        spad = -(-s // 128) * 128
        if not final:
            assert s % 128 == 0
        fa = min(fa, M - n8c)
        ra = min(ra, M - n8c)
        wf = M - (r0 + 2) // 128 * 128
        wr = -(-(510 - r0) // 128) * 128
        w = min(M, max(wf, wr))
        chunks.append(dict(p=p, s=s, spad=spad, n8=n8c, w=w,
                           fwd_start=fa, flip_start=ra, final=final))
        p += s
    tot = sum(ch["spad"] for ch in chunks)
    tabs = np.zeros((2, 2, 1, tot), np.int32)
    ioff = 0
    for ch in chunks:
        g = _GIDX[ch["p"]: ch["p"] + ch["s"]]
        r, c = g // M, g % M
        sl = slice(ioff, ioff + ch["s"])
        tabs[1, 0, 0, sl] = r - ch["fwd_start"]
        tabs[1, 1, 0, sl] = c - (M - ch["w"])
        tabs[0, 0, 0, sl] = (511 - r) - ch["flip_start"]
        tabs[0, 1, 0, sl] = 511 - c
        assert tabs[1, 1, 0, sl].min() >= 0 and tabs[0, 1, 0, sl].max() < ch["w"]
        ch["idx_off"] = ioff
        ioff += ch["spad"]
    assert tabs[:, 0].max() < ROWCAP8 and tabs.min() >= 0
    return chunks, tabs


_CHUNKS, _TABS = _build_plan()
_NCH = len(_CHUNKS)

NC, NS = 2, 16
NW = NC * NS
PAIRS_PER_W = BC // NW


def _sc_body(in_ref, idx_ref, flags_ref, out_ref,
             win_v, ri_v, ci_v, out8_v, flag_v, sems, isems, osem):
    wid = lax.axis_index("s") * NC + lax.axis_index("c")
    b = wid // NS
    ch0 = (wid % NS) * PAIRS_PER_W

    pltpu.sync_copy(flags_ref, flag_v)
    fv = flag_v[...]
    lane = lax.broadcasted_iota(jnp.int32, (16,), 0)
    flag = jnp.sum(jnp.where(lane == b, fv, 0))

    prev_out = [None]

    for ch in _CHUNKS:
        spad, s, n8 = ch["spad"], ch["s"], ch["n8"]
        idx_cp = [
            pltpu.make_async_copy(
                idx_ref.at[flag, t, 0, pl.ds(ch["idx_off"], spad)],
                (ri_v if t == 0 else ci_v).at[pl.ds(0, spad)],
                isems.at[t])
            for t in range(2)
        ]
        for cp in idx_cp:
            cp.start()
        rs = pl.multiple_of(
            flag * ch["fwd_start"] + (1 - flag) * ch["flip_start"], 8)
        w = ch["w"]
        c0 = pl.multiple_of(flag * (M - w), 128)

        def start_win(k, par, n8=n8, rs=rs, w=w, c0=c0):
            pltpu.make_async_copy(
                in_ref.at[b, ch0 + k, pl.ds(rs, n8), pl.ds(c0, w)],
                win_v.at[par, pl.ds(0, n8), pl.ds(0, w)],
                sems.at[par]).start()

        start_win(0, 0)
        for cp in idx_cp:
            cp.wait()
        if prev_out[0] is not None:
            prev_out[0].wait()

        def chan_body(k, carry, n8=n8, spad=spad, rs=rs, w=w, c0=c0):
            par = lax.rem(k, 2)
            pltpu.make_async_copy(
                in_ref.at[b, ch0 + k, pl.ds(rs, n8), pl.ds(c0, w)],
                win_v.at[par, pl.ds(0, n8), pl.ds(0, w)],
                sems.at[par]).wait()

            @pl.when(k < PAIRS_PER_W - 1)
            def _():
                start_win(k + 1, 1 - par)

            win = win_v.at[par]

            @plsc.parallel_loop(0, spad, step=16, unroll=16)
            def gbody(j, k=k):
                riv = ri_v[pl.ds(j, 16)]
                civ = ci_v[pl.ds(j, 16)]
                out8_v[k, pl.ds(j, 16)] = plsc.load_gather(win, [riv, civ])

            return carry

        lax.fori_loop(0, PAIRS_PER_W, chan_body, 0)
        if ch["final"]:
            pstart = pl.multiple_of(ch["p"] + 0 * flag, 128)
        else:
            pstart = ch["p"]
        ocp = pltpu.make_async_copy(
            out8_v.at[:, pl.ds(0, spad)],
            out_ref.at[b, pl.ds(ch0, 8), pl.ds(pstart, spad)],
            osem)
        ocp.start()
        prev_out[0] = ocp
    prev_out[0].wait()


JB = 1024
_NJB = -(-OUTLEN // JB)


def _tc_transpose_body(y_ref, z_ref):
    x = y_ref[...].reshape(B * C, JB)
    z_ref[...] = x.T.reshape(JB, B, C)


def _to_channel_minor(y):
    z = pl.pallas_call(
        _tc_transpose_body,
        grid=(_NJB,),
        in_specs=[pl.BlockSpec((B, C, JB), lambda j: (0, 0, j))],
        out_specs=pl.BlockSpec((JB, B, C), lambda j: (j, 0, 0)),
        out_shape=jax.ShapeDtypeStruct((OUTLEN, B, C), jnp.float32),
        compiler_params=pltpu.CompilerParams(
            dimension_semantics=("arbitrary",)),
    )(y)
    return jnp.transpose(z, (1, 2, 0))


def kernel(inputs, reverse_complement_flags):
    flags16 = jnp.zeros((16,), jnp.int32).at[:B].set(
        reverse_complement_flags.astype(jnp.int32))
    idx_tab = jnp.asarray(_TABS)

    mesh = plsc.VectorSubcoreMesh(core_axis_name="c", subcore_axis_name="s",
                                  num_cores=NC, num_subcores=NS)
    fn = pl.kernel(
        _sc_body,
        out_type=jax.ShapeDtypeStruct((B, C, OUTLEN), jnp.float32),
        mesh=mesh,
        scratch_types=[
            pltpu.VMEM((2, ROWCAP8, M), jnp.float32),
            pltpu.VMEM((OUTCAP,), jnp.int32),
            pltpu.VMEM((OUTCAP,), jnp.int32),
            pltpu.VMEM((PAIRS_PER_W, OUTCAP), jnp.float32),
            pltpu.VMEM((16,), jnp.int32),
            pltpu.SemaphoreType.DMA((2,)),
            pltpu.SemaphoreType.DMA((2,)),
            pltpu.SemaphoreType.DMA,
        ],
        compiler_params=pltpu.CompilerParams(disable_bounds_checks=True,
                                             needs_layout_passes=False),
    )
    return _to_channel_minor(fn(inputs, idx_tab, flags16))
